# Initial kernel scaffold; baseline (speedup 1.0000x reference)
import jax
import jax.numpy as jnp
from jax.experimental import pallas as pl


def kernel(x_ip, x_con, ei_ip_ip, ei_con_src, ei_con_dst, ei_ip_con, ei_con_ip, Wl, Wr, bl):
    raise NotImplementedError("write your pallas kernel here")


# speedup vs baseline: 1.0182x; 1.0182x over previous
import functools

import jax
import jax.numpy as jnp
from jax import lax
from jax.experimental import pallas as pl

N_IP = 50000
N_CON = 100000
D = 128

_BR = 2000


def _dense_body(k_branches, *refs):
    nb = k_branches
    summed = [refs[2 * i] for i in range(nb)]
    cnt = [refs[2 * i + 1] for i in range(nb)]
    x_ref = refs[2 * nb]
    w = refs[2 * nb + 1: 2 * nb + 1 + 3 * nb]
    out_ref = refs[-1]
    x = x_ref[...]
    acc = None
    for i in range(nb):
        wl, bl, wr = w[3 * i], w[3 * i + 1], w[3 * i + 2]
        c = cnt[i][...]
        mean = summed[i][...] / jnp.maximum(c, 1.0)
        o = lax.dot_general(mean, wl[...], (((1,), (1,)), ((), ())),
                            preferred_element_type=jnp.float32)
        o = o + bl[...] + lax.dot_general(x, wr[...], (((1,), (1,)), ((), ())),
                                          preferred_element_type=jnp.float32)
        nrm = jnp.sqrt(jnp.sum(o * o, axis=-1, keepdims=True))
        o = o / jnp.maximum(nrm, 1e-12)
        acc = o if acc is None else acc + o
    out_ref[...] = jnp.where(acc >= 0, acc, 0.01 * acc)


def _dense_stage(summed_list, cnt_list, x_dst, wl_list, bl_list, wr_list):
    nb = len(summed_list)
    n = x_dst.shape[0]
    grid = (n // _BR,) if n % _BR == 0 else (pl.cdiv(n, _BR),)
    row_spec = pl.BlockSpec((_BR, D), lambda i: (i, 0))
    cnt_spec = pl.BlockSpec((_BR, 1), lambda i: (i, 0))
    w_spec = pl.BlockSpec((D, D), lambda i: (0, 0))
    b_spec = pl.BlockSpec((1, D), lambda i: (0, 0))
    in_specs = []
    args = []
    for s, c in zip(summed_list, cnt_list):
        in_specs += [row_spec, cnt_spec]
        args += [s, c.reshape(n, 1)]
    in_specs.append(row_spec)
    args.append(x_dst)
    for wl, bl, wr in zip(wl_list, bl_list, wr_list):
        in_specs += [w_spec, b_spec, w_spec]
        args += [wl, bl.reshape(1, D), wr]
    return pl.pallas_call(
        functools.partial(_dense_body, nb),
        grid=grid,
        in_specs=in_specs,
        out_specs=row_spec,
        out_shape=jax.ShapeDtypeStruct((n, D), jnp.float32),
    )(*args)


def _agg(x_src, ei, n_dst):
    msg = jnp.take(x_src, ei[0], axis=0)
    summed = jax.ops.segment_sum(msg, ei[1], num_segments=n_dst)
    cnt = jax.ops.segment_sum(jnp.ones((ei.shape[1],), jnp.float32), ei[1],
                              num_segments=n_dst)
    return summed, cnt


def kernel(x_ip, x_con, ei_ip_ip, ei_con_src, ei_con_dst, ei_ip_con, ei_con_ip, Wl, Wr, bl):
    for idx in (0, 5):
        s_ip, c_ip = _agg(x_ip, ei_ip_ip, N_IP)
        s_c1, c_c1 = _agg(x_con, ei_con_src, N_CON)
        s_c2, c_c2 = _agg(x_con, ei_con_dst, N_CON)
        o_ip = _dense_stage([s_ip], [c_ip], x_ip, [Wl[idx]], [bl[idx]], [Wr[idx]])
        o_con = _dense_stage([s_c1, s_c2], [c_c1, c_c2], x_con,
                             [Wl[idx + 1], Wl[idx + 2]], [bl[idx + 1], bl[idx + 2]],
                             [Wr[idx + 1], Wr[idx + 2]])
        x_ip, x_con = o_ip, o_con
        s_con, c_con = _agg(x_ip, ei_ip_con, N_CON)
        s_ip2, c_ip2 = _agg(x_con, ei_con_ip, N_IP)
        x_con = _dense_stage([s_con], [c_con], x_con, [Wl[idx + 3]], [bl[idx + 3]], [Wr[idx + 3]])
        x_ip = _dense_stage([s_ip2], [c_ip2], x_ip, [Wl[idx + 4]], [bl[idx + 4]], [Wr[idx + 4]])
    return (x_ip, x_con)


# speedup vs baseline: 1.1548x; 1.1341x over previous
import functools

import jax
import jax.numpy as jnp
from jax import lax
from jax.experimental import pallas as pl
from jax.experimental.pallas import tpu as pltpu
from jax.experimental.pallas import tpu_sc as plsc

N_IP = 50000
N_CON = 100000
D = 128
NF = 8
_G = 128
_IB = 16
_NT = 16
_NW = 32
_BR = 2000


def _stripe(n_dst):
    return -(-(n_dst + 128) // (_NT * 8)) * 8


def _pad128(n):
    return -(-n // 128) * 128


_MAX_STRIPE = _stripe(N_CON)


def _one_segsum(n_dst, ng, c, s, xflat, src_all, dst_all, out,
                src_blk, dst_blk, stag, zeros, acc, sem):
    stripe = _stripe(n_dst)
    rd = _pad128(n_dst) // _NT
    w = c * _NT + s
    nf = NF if xflat is not None else 1
    blocks = [(i * _IB, _IB) for i in range(ng // _IB)]
    if ng % _IB:
        blocks.append((ng - ng % _IB, ng % _IB))
    for f in range(nf):
        pltpu.sync_copy(zeros.at[pl.ds(0, stripe)],
                        acc.at[pl.ds(s * stripe, stripe)])
        plsc.subcore_barrier()
        for b0, bs in blocks:
            pltpu.sync_copy(dst_all.at[w, pl.ds(b0, bs)],
                            dst_blk.at[pl.ds(0, bs)])
            if xflat is not None:
                pltpu.sync_copy(src_all.at[f, w, pl.ds(b0, bs)],
                                src_blk.at[pl.ds(0, bs)])

                def grp(g, carry):
                    pltpu.async_copy(xflat.at[src_blk.at[g]], stag, sem).wait()
                    pltpu.sync_copy(stag, acc.at[dst_blk.at[g]], add=True)
                    return carry
            else:
                def grp(g, carry):
                    pltpu.sync_copy(stag, acc.at[dst_blk.at[g]], add=True)
                    return carry
            lax.fori_loop(0, bs, grp, 0)
        plsc.subcore_barrier()
        if xflat is not None:
            pltpu.sync_copy(acc.at[pl.ds(s * rd, rd)],
                            out.at[c, pl.ds(s * rd, rd), f])
        else:
            pltpu.sync_copy(acc.at[pl.ds(s * rd, rd)],
                            out.at[c, pl.ds(s * rd, rd)])
        plsc.subcore_barrier()


@functools.cache
def _phase_kernel(specs, counts):
    max_rows = max(_stripe(n) for n, _ in specs) * _NT
    mesh = plsc.VectorSubcoreMesh(core_axis_name="c", subcore_axis_name="s")
    out_type = tuple(
        jax.ShapeDtypeStruct((2, _pad128(n), 16) if counts
                             else (2, _pad128(n), NF, 16), jnp.float32)
        for n, _ in specs)

    @functools.partial(
        pl.kernel, mesh=mesh,
        out_type=out_type,
        compiler_params=pltpu.CompilerParams(use_tc_tiling_on_sc=False),
        scratch_types=[
            pltpu.VMEM((_IB, _G), jnp.int32),
            pltpu.VMEM((_IB, _G), jnp.int32),
            pltpu.VMEM((_G, 16), jnp.float32),
            pltpu.VMEM_SHARED((max_rows, 16), jnp.float32),
            pltpu.SemaphoreType.DMA,
        ],
    )
    def k(*refs):
        nseg = len(specs)
        nin = (nseg if counts else 3 * nseg) + 1
        ins = refs[:nin - 1]
        zeros = refs[nin - 1]
        outs = refs[nin:nin + nseg]
        src_blk, dst_blk, stag, acc, sem = refs[nin + nseg:nin + nseg + 5]
        c = lax.axis_index("c")
        s = lax.axis_index("s")
        if counts:
            pltpu.sync_copy(zeros.at[pl.ds(_MAX_STRIPE, _G)], stag)
        for i, (n_dst, ng) in enumerate(specs):
            if counts:
                _one_segsum(n_dst, ng, c, s, None, None, ins[i], outs[i],
                            src_blk, dst_blk, stag, zeros, acc, sem)
            else:
                _one_segsum(n_dst, ng, c, s, ins[3 * i], ins[3 * i + 1],
                            ins[3 * i + 2], outs[i],
                            src_blk, dst_blk, stag, zeros, acc, sem)

    return k


def _dense_body(nb, *refs):
    summed = [refs[2 * i] for i in range(nb)]
    cnts = [refs[2 * i + 1] for i in range(nb)]
    x_ref = refs[2 * nb]
    w = refs[2 * nb + 1: 2 * nb + 1 + 3 * nb]
    out_ref = refs[-1]
    x = x_ref[...]
    acc = None
    for i in range(nb):
        sp = summed[i][...]
        mean = sp[0] + sp[1]
        cp = cnts[i][...]
        cnt = (cp[0] + cp[1])[:, 0:1]
        mean = mean / jnp.maximum(cnt, 1.0)
        wl, bl_, wr = w[3 * i], w[3 * i + 1], w[3 * i + 2]
        o = lax.dot_general(mean, wl[...], (((1,), (1,)), ((), ())),
                            preferred_element_type=jnp.float32)
        o = o + bl_[...] + lax.dot_general(x, wr[...], (((1,), (1,)), ((), ())),
                                           preferred_element_type=jnp.float32)
        nrm = jnp.sqrt(jnp.sum(o * o, axis=-1, keepdims=True))
        o = o / jnp.maximum(nrm, 1e-12)
        acc = o if acc is None else acc + o
    res = jnp.where(acc >= 0, acc, 0.01 * acc)
    out_ref[...] = res


def _dense_stage(summed_list, cnt_list, x_dst, wl_list, bl_list, wr_list):
    nb = len(summed_list)
    n = x_dst.shape[0]
    grid = (n // _BR,)
    row_spec = pl.BlockSpec((_BR, D), lambda i: (i, 0))
    sum_spec = pl.BlockSpec((2, _BR, D), lambda i: (0, i, 0))
    cnt_spec = pl.BlockSpec((2, _BR, 16), lambda i: (0, i, 0))
    w_spec = pl.BlockSpec((D, D), lambda i: (0, 0))
    b_spec = pl.BlockSpec((1, D), lambda i: (0, 0))
    in_specs = []
    args = []
    for s, c in zip(summed_list, cnt_list):
        in_specs += [sum_spec, cnt_spec]
        args += [s.reshape(2, s.shape[1], D), c]
    in_specs.append(row_spec)
    args.append(x_dst)
    for wl, bl_, wr in zip(wl_list, bl_list, wr_list):
        in_specs += [w_spec, b_spec, w_spec]
        args += [wl, bl_.reshape(1, D), wr]
    return pl.pallas_call(
        functools.partial(_dense_body, nb),
        grid=grid,
        in_specs=in_specs,
        out_specs=row_spec,
        out_shape=jax.ShapeDtypeStruct((n, D), jnp.float32),
    )(*args)


def _prep_edges(ei, n_src, n_dst):
    e = ei.shape[1]
    ng = -(-(-(-e // _NW)) // _G)
    ng = -(-ng // 8) * 8
    pad = _NW * ng * _G - e
    ar = jnp.arange(pad, dtype=jnp.int32)
    src = jnp.concatenate([ei[0], ar % jnp.int32(n_src)]).reshape(_NW, ng, _G)
    dst = jnp.concatenate([ei[1], jnp.int32(n_dst) + (ar % 128)]).reshape(_NW, ng, _G)
    offs = jnp.arange(NF, dtype=jnp.int32)[:, None, None, None]
    return src[None] * NF + offs, dst, ng


def _flat(x):
    return x.reshape(x.shape[0] * NF, 16)


def kernel(x_ip, x_con, ei_ip_ip, ei_con_src, ei_con_dst, ei_ip_con, ei_con_ip, Wl, Wr, bl):
    src_ii, dst_ii, ng_ii = _prep_edges(ei_ip_ip, N_IP, N_IP)
    src_cs, dst_cs, ng_c = _prep_edges(ei_con_src, N_CON, N_CON)
    src_cd, dst_cd, _ = _prep_edges(ei_con_dst, N_CON, N_CON)
    src_ic, dst_ic, _ = _prep_edges(ei_ip_con, N_IP, N_CON)
    src_ci, dst_ci, _ = _prep_edges(ei_con_ip, N_CON, N_IP)

    zc = jnp.concatenate([jnp.zeros((_MAX_STRIPE, 16), jnp.float32),
                          jnp.ones((_G, 16), jnp.float32)], axis=0)

    cnt_specs = ((N_IP, ng_ii), (N_CON, ng_c), (N_CON, ng_c),
                 (N_CON, ng_c), (N_IP, ng_c))
    cnt_ii, cnt_cs, cnt_cd, cnt_ic, cnt_ci = _phase_kernel(
        cnt_specs, True)(dst_ii, dst_cs, dst_cd, dst_ic, dst_ci, zc)

    t_specs = ((N_IP, ng_ii), (N_CON, ng_c), (N_CON, ng_c))
    s_specs = ((N_CON, ng_c), (N_IP, ng_c))
    temporal = _phase_kernel(t_specs, False)
    spatial = _phase_kernel(s_specs, False)

    x_ip, _ = lax.optimization_barrier((x_ip, cnt_ci))

    for idx in (0, 5):
        s_ii, s_cs, s_cd = temporal(
            _flat(x_ip), src_ii, dst_ii,
            _flat(x_con), src_cs, dst_cs,
            _flat(x_con), src_cd, dst_cd, zc)
        o_ip = _dense_stage([s_ii], [cnt_ii], x_ip,
                            [Wl[idx]], [bl[idx]], [Wr[idx]])
        o_con = _dense_stage([s_cs, s_cd], [cnt_cs, cnt_cd], x_con,
                             [Wl[idx + 1], Wl[idx + 2]],
                             [bl[idx + 1], bl[idx + 2]],
                             [Wr[idx + 1], Wr[idx + 2]])
        s_ic, s_ci = spatial(
            _flat(o_ip), src_ic, dst_ic,
            _flat(o_con), src_ci, dst_ci, zc)
        x_con = _dense_stage([s_ic], [cnt_ic], o_con,
                             [Wl[idx + 3]], [bl[idx + 3]], [Wr[idx + 3]])
        x_ip = _dense_stage([s_ci], [cnt_ci], o_ip,
                            [Wl[idx + 4]], [bl[idx + 4]], [Wr[idx + 4]])
    return (x_ip, x_con)


# speedup vs baseline: 1.9019x; 1.6469x over previous
import functools

import jax
import jax.numpy as jnp
from jax import lax
from jax.experimental import pallas as pl
from jax.experimental.pallas import tpu as pltpu
from jax.experimental.pallas import tpu_sc as plsc

N_IP = 50000
N_CON = 100000
D = 128
NF = 8
_G = 128
_IB = 16
_NT = 16
_NW = 32
_BR = 2000


def _stripe(n_dst):
    return -(-(n_dst + 128) // (_NT * 8)) * 8


def _pad128(n):
    return -(-n // 128) * 128


_MAX_STRIPE = _stripe(N_CON)


_K = 4
_M = 8


def _one_segsum(n_dst, ng, c, s, xflat, src_all, dst_all, out,
                src_blk, dst_blk, stag, zeros, acc,
                gsem, ssem, isem_s, isem_d):
    stripe = _stripe(n_dst)
    rd = _pad128(n_dst) // _NT
    w = c * _NT + s
    nf = NF if xflat is not None else 1
    nblk16 = ng // _IB

    def _drain(sem, ref):
        pltpu.make_async_copy(zeros.at[pl.ds(0, ref.shape[0])], ref, sem).wait()

    for f in range(nf):
        pltpu.sync_copy(zeros.at[pl.ds(0, stripe)],
                        acc.at[pl.ds(s * stripe, stripe)])
        plsc.subcore_barrier()

        if xflat is None:
            def cblk(b, carry):
                pltpu.sync_copy(dst_all.at[w, pl.ds(b * _IB, _IB)],
                                dst_blk.at[0])

                def grp(g, carry2):
                    pltpu.async_copy(stag.at[0],
                                     acc.at[dst_blk.at[0, g]], ssem[0],
                                     add=True)
                    return carry2
                lax.fori_loop(0, _IB, grp, 0)

                def dr(g, carry2):
                    _drain(ssem[0], stag.at[0])
                    return carry2
                lax.fori_loop(0, _IB, dr, 0)
                return carry
            lax.fori_loop(0, nblk16, cblk, 0)
        else:
            pltpu.sync_copy(src_all.at[f, w, pl.ds(0, _IB)], src_blk.at[0])
            pltpu.sync_copy(dst_all.at[w, pl.ds(0, _IB)], dst_blk.at[0])
            for r in range(_K):
                pltpu.async_copy(xflat.at[src_blk.at[0, r]], stag.at[r],
                                 gsem[r])

            def blk(j, carry):
                buf = lax.rem(lax.div(j, 2), 2)
                half = lax.rem(j, 2)
                blk16 = lax.div(j, 2)
                for r in range(_M):
                    g = j * _M + r
                    row = half * _M + r
                    if r == 0:
                        @pl.when((half == 1) & (blk16 + 1 < nblk16))
                        def _():
                            nb16 = blk16 + 1
                            pltpu.async_copy(
                                src_all.at[f, w, pl.ds(nb16 * _IB, _IB)],
                                src_blk.at[1 - buf], isem_s)
                            pltpu.async_copy(
                                dst_all.at[w, pl.ds(nb16 * _IB, _IB)],
                                dst_blk.at[1 - buf], isem_d)
                    if r == _K:
                        @pl.when((half == 1) & (blk16 + 1 < nblk16))
                        def _():
                            _drain(isem_s, stag.at[0])
                            _drain(isem_d, stag.at[0])
                    _drain(gsem[r], stag.at[r])
                    pltpu.async_copy(stag.at[r], acc.at[dst_blk.at[buf, row]],
                                     ssem[r], add=True)
                    nxt = g + _K
                    rn = (r + _K) % _M

                    @pl.when(nxt < ng)
                    def _():
                        @pl.when(nxt >= _M)
                        def _():
                            _drain(ssem[rn], stag.at[rn])
                        buf_n = lax.rem(lax.div(nxt, _IB), 2)
                        row_n = lax.rem(nxt, _IB)
                        pltpu.async_copy(xflat.at[src_blk.at[buf_n, row_n]],
                                         stag.at[rn], gsem[rn])
                return carry
            lax.fori_loop(0, ng // _M, blk, 0)
            for r in range(_M):
                _drain(ssem[r], stag.at[r])

        plsc.subcore_barrier()
        if xflat is not None:
            pltpu.sync_copy(acc.at[pl.ds(s * rd, rd)],
                            out.at[c, pl.ds(s * rd, rd), f])
        else:
            pltpu.sync_copy(acc.at[pl.ds(s * rd, rd)],
                            out.at[c, pl.ds(s * rd, rd)])
        plsc.subcore_barrier()


@functools.cache
def _phase_kernel(specs, counts):
    max_rows = max(_stripe(n) for n, _ in specs) * _NT
    mesh = plsc.VectorSubcoreMesh(core_axis_name="c", subcore_axis_name="s")
    out_type = tuple(
        jax.ShapeDtypeStruct((2, _pad128(n), 16) if counts
                             else (2, _pad128(n), NF, 16), jnp.float32)
        for n, _ in specs)

    @functools.partial(
        pl.kernel, mesh=mesh,
        out_type=out_type,
        compiler_params=pltpu.CompilerParams(use_tc_tiling_on_sc=False),
        scratch_types=[
            pltpu.VMEM((2, _IB, _G), jnp.int32),
            pltpu.VMEM((2, _IB, _G), jnp.int32),
            pltpu.VMEM((_M, _G, 16), jnp.float32),
            pltpu.VMEM_SHARED((max_rows, 16), jnp.float32),
        ] + [pltpu.SemaphoreType.DMA] * (2 * _M + 2),
    )
    def k(*refs):
        nseg = len(specs)
        nin = (nseg if counts else 3 * nseg) + 1
        ins = refs[:nin - 1]
        zeros = refs[nin - 1]
        outs = refs[nin:nin + nseg]
        src_blk, dst_blk, stag, acc = refs[nin + nseg:nin + nseg + 4]
        sems = refs[nin + nseg + 4:]
        gsem = sems[:_M]
        ssem = sems[_M:2 * _M]
        isem_s, isem_d = sems[2 * _M], sems[2 * _M + 1]
        c = lax.axis_index("c")
        s = lax.axis_index("s")
        if counts:
            pltpu.sync_copy(zeros.at[pl.ds(_MAX_STRIPE, _G)], stag.at[0])
        for i, (n_dst, ng) in enumerate(specs):
            if counts:
                _one_segsum(n_dst, ng, c, s, None, None, ins[i], outs[i],
                            src_blk, dst_blk, stag, zeros, acc,
                            gsem, ssem, isem_s, isem_d)
            else:
                _one_segsum(n_dst, ng, c, s, ins[3 * i], ins[3 * i + 1],
                            ins[3 * i + 2], outs[i],
                            src_blk, dst_blk, stag, zeros, acc,
                            gsem, ssem, isem_s, isem_d)

    return k


def _dense_body(nb, *refs):
    summed = [refs[2 * i] for i in range(nb)]
    cnts = [refs[2 * i + 1] for i in range(nb)]
    x_ref = refs[2 * nb]
    w = refs[2 * nb + 1: 2 * nb + 1 + 3 * nb]
    out_ref = refs[-1]
    x = x_ref[...]
    acc = None
    for i in range(nb):
        sp = summed[i][...]
        mean = sp[0] + sp[1]
        cp = cnts[i][...]
        cnt = (cp[0] + cp[1])[:, 0:1]
        mean = mean / jnp.maximum(cnt, 1.0)
        wl, bl_, wr = w[3 * i], w[3 * i + 1], w[3 * i + 2]
        o = lax.dot_general(mean, wl[...], (((1,), (1,)), ((), ())),
                            preferred_element_type=jnp.float32)
        o = o + bl_[...] + lax.dot_general(x, wr[...], (((1,), (1,)), ((), ())),
                                           preferred_element_type=jnp.float32)
        nrm = jnp.sqrt(jnp.sum(o * o, axis=-1, keepdims=True))
        o = o / jnp.maximum(nrm, 1e-12)
        acc = o if acc is None else acc + o
    res = jnp.where(acc >= 0, acc, 0.01 * acc)
    out_ref[...] = res


def _dense_stage(summed_list, cnt_list, x_dst, wl_list, bl_list, wr_list):
    nb = len(summed_list)
    n = x_dst.shape[0]
    grid = (n // _BR,)
    row_spec = pl.BlockSpec((_BR, D), lambda i: (i, 0))
    sum_spec = pl.BlockSpec((2, _BR, D), lambda i: (0, i, 0))
    cnt_spec = pl.BlockSpec((2, _BR, 16), lambda i: (0, i, 0))
    w_spec = pl.BlockSpec((D, D), lambda i: (0, 0))
    b_spec = pl.BlockSpec((1, D), lambda i: (0, 0))
    in_specs = []
    args = []
    for s, c in zip(summed_list, cnt_list):
        in_specs += [sum_spec, cnt_spec]
        args += [s.reshape(2, s.shape[1], D), c]
    in_specs.append(row_spec)
    args.append(x_dst)
    for wl, bl_, wr in zip(wl_list, bl_list, wr_list):
        in_specs += [w_spec, b_spec, w_spec]
        args += [wl, bl_.reshape(1, D), wr]
    return pl.pallas_call(
        functools.partial(_dense_body, nb),
        grid=grid,
        in_specs=in_specs,
        out_specs=row_spec,
        out_shape=jax.ShapeDtypeStruct((n, D), jnp.float32),
    )(*args)


def _prep_edges(ei, n_src, n_dst):
    e = ei.shape[1]
    ng = -(-(-(-e // _NW)) // _G)
    ng = -(-ng // _IB) * _IB
    pad = _NW * ng * _G - e
    ar = jnp.arange(pad, dtype=jnp.int32)
    src = jnp.concatenate([ei[0], ar % jnp.int32(n_src)]).reshape(_NW, ng, _G)
    dst = jnp.concatenate([ei[1], jnp.int32(n_dst) + (ar % 128)]).reshape(_NW, ng, _G)
    offs = jnp.arange(NF, dtype=jnp.int32)[:, None, None, None]
    return src[None] * NF + offs, dst, ng


def _flat(x):
    return x.reshape(x.shape[0] * NF, 16)


def kernel(x_ip, x_con, ei_ip_ip, ei_con_src, ei_con_dst, ei_ip_con, ei_con_ip, Wl, Wr, bl):
    src_ii, dst_ii, ng_ii = _prep_edges(ei_ip_ip, N_IP, N_IP)
    src_cs, dst_cs, ng_c = _prep_edges(ei_con_src, N_CON, N_CON)
    src_cd, dst_cd, _ = _prep_edges(ei_con_dst, N_CON, N_CON)
    src_ic, dst_ic, _ = _prep_edges(ei_ip_con, N_IP, N_CON)
    src_ci, dst_ci, _ = _prep_edges(ei_con_ip, N_CON, N_IP)

    zc = jnp.concatenate([jnp.zeros((_MAX_STRIPE, 16), jnp.float32),
                          jnp.ones((_G, 16), jnp.float32)], axis=0)

    cnt_specs = ((N_IP, ng_ii), (N_CON, ng_c), (N_CON, ng_c),
                 (N_CON, ng_c), (N_IP, ng_c))
    cnt_ii, cnt_cs, cnt_cd, cnt_ic, cnt_ci = _phase_kernel(
        cnt_specs, True)(dst_ii, dst_cs, dst_cd, dst_ic, dst_ci, zc)

    t_specs = ((N_IP, ng_ii), (N_CON, ng_c), (N_CON, ng_c))
    s_specs = ((N_CON, ng_c), (N_IP, ng_c))
    temporal = _phase_kernel(t_specs, False)
    spatial = _phase_kernel(s_specs, False)

    x_ip, _ = lax.optimization_barrier((x_ip, cnt_ci))

    for idx in (0, 5):
        s_ii, s_cs, s_cd = temporal(
            _flat(x_ip), src_ii, dst_ii,
            _flat(x_con), src_cs, dst_cs,
            _flat(x_con), src_cd, dst_cd, zc)
        o_ip = _dense_stage([s_ii], [cnt_ii], x_ip,
                            [Wl[idx]], [bl[idx]], [Wr[idx]])
        o_con = _dense_stage([s_cs, s_cd], [cnt_cs, cnt_cd], x_con,
                             [Wl[idx + 1], Wl[idx + 2]],
                             [bl[idx + 1], bl[idx + 2]],
                             [Wr[idx + 1], Wr[idx + 2]])
        s_ic, s_ci = spatial(
            _flat(o_ip), src_ic, dst_ic,
            _flat(o_con), src_ci, dst_ci, zc)
        x_con = _dense_stage([s_ic], [cnt_ic], o_con,
                             [Wl[idx + 3]], [bl[idx + 3]], [Wr[idx + 3]])
        x_ip = _dense_stage([s_ci], [cnt_ci], o_ip,
                            [Wl[idx + 4]], [bl[idx + 4]], [Wr[idx + 4]])
    return (x_ip, x_con)
